# baseline (speedup 1.0000x reference)
import jax
import jax.numpy as jnp
from jax import lax
from jax.experimental import pallas as pl
from jax.experimental.pallas import tpu as pltpu
from jax.experimental.pallas import tpu_sc as plsc

MULT = 1315423911
NQ = 4
DQ = 112
LQ = 128
L = 16


def _scale_body(emb_ref, g_ref, out_ref):
    sig = jax.nn.sigmoid(g_ref[...])
    for q in range(NQ):
        d0 = q * DQ
        out_ref[q, :, :DQ] = emb_ref[:, d0:d0 + DQ] * sig[:, d0:d0 + DQ]
        out_ref[q, :, DQ:] = jnp.zeros_like(out_ref[q, :, DQ:])


def _scale_table(emb, g):
    V, D = emb.shape
    blk = 512
    return pl.pallas_call(
        _scale_body,
        out_shape=jax.ShapeDtypeStruct((NQ, V, LQ), emb.dtype),
        grid=(V // blk,),
        in_specs=[
            pl.BlockSpec((blk, D), lambda i: (i, 0)),
            pl.BlockSpec((1, D), lambda i: (0, 0)),
        ],
        out_specs=pl.BlockSpec((NQ, blk, LQ), lambda i: (0, i, 0)),
    )(emb, g.reshape(1, D))


def _make_sc_lookup(B, T, V, D):
    info = plsc.get_sparse_core_info()
    NW = info.num_cores * info.num_subcores
    NB = B // 128
    assert NB * NQ == NW and D == NQ * DQ
    NRING = 4
    NSLAB = 2
    assert T % NRING == 0
    n_outer = T // NRING

    mesh = plsc.VectorSubcoreMesh(core_axis_name="c", subcore_axis_name="s")

    def body(xt_hbm, emb4_hbm, out_hbm, xs, ring, slabs, gsems, osems):
        wid = lax.axis_index("c") * info.num_subcores + lax.axis_index("s")
        j = wid & (NB - 1)
        q = lax.shift_right_logical(wid, 3)
        b0 = j * 128

        pltpu.sync_copy(xt_hbm.at[:, pl.ds(b0, 128)], xs)

        def hback(i, _):
            t = (T - 1) - i
            for m in range(128 // L):
                cur = xs[t, pl.ds(m * L, L)]
                prv = xs[t - 1, pl.ds(m * L, L)]
                xs[t, pl.ds(m * L, L)] = (prv * MULT + cur) & (V - 1)
            return 0

        lax.fori_loop(0, T - 1, hback, 0)
        for m in range(128 // L):
            xs[0, pl.ds(m * L, L)] = xs[0, pl.ds(m * L, L)] & (V - 1)

        def issue_gather(t, s):
            return pltpu.async_copy(
                emb4_hbm.at[q].at[xs.at[t]], ring[s], gsems[s]
            )

        for s in range(NRING):
            issue_gather(s, s)

        idx_b = [m * L + lax.iota(jnp.int32, L) for m in range(128 // L)]

        def outer(p, _):
            for s in range(NRING):
                t = p * NRING + s
                o = s % NSLAB
                pltpu.make_async_copy(
                    emb4_hbm.at[0], ring[s], gsems[s]
                ).wait()

                @pl.when(t >= NSLAB)
                def _():
                    pltpu.make_async_copy(
                        slabs[o], out_hbm.at[0, pl.ds(0, DQ), pl.ds(0, 128)],
                        osems[o],
                    ).wait()

                def drow(d, _2):
                    dd = jnp.full((L,), d, jnp.int32)
                    for m in range(128 // L):
                        v = plsc.load_gather(ring[s], [idx_b[m], dd])
                        slabs[o][d, pl.ds(m * L, L)] = v
                    return 0

                lax.fori_loop(0, DQ, drow, 0)

                pltpu.async_copy(
                    slabs[o],
                    out_hbm.at[t, pl.ds(q * DQ, DQ), pl.ds(b0, 128)],
                    osems[o],
                )

                @pl.when(t + NRING < T)
                def _():
                    issue_gather(t + NRING, s)

            return 0

        lax.fori_loop(0, n_outer, outer, 0)

        for o in range(NSLAB):
            pltpu.make_async_copy(
                slabs[o], out_hbm.at[0, pl.ds(0, DQ), pl.ds(0, 128)], osems[o]
            ).wait()

    scratch = [
        pltpu.VMEM((T, 128), jnp.int32),
        [pltpu.VMEM((128, LQ), jnp.float32) for _ in range(NRING)],
        [pltpu.VMEM((DQ, 128), jnp.float32) for _ in range(NSLAB)],
        [pltpu.SemaphoreType.DMA for _ in range(NRING)],
        [pltpu.SemaphoreType.DMA for _ in range(NSLAB)],
    ]

    return pl.kernel(
        body,
        out_type=jax.ShapeDtypeStruct((T, D, B), jnp.float32),
        mesh=mesh,
        scratch_types=scratch,
        compiler_params=pltpu.CompilerParams(needs_layout_passes=False),
    )


@jax.jit
def kernel(x, emb, g):
    B, T = x.shape
    V, D = emb.shape
    emb4 = _scale_table(emb, g)
    lookup = _make_sc_lookup(B, T, V, D)
    out_t = lookup(x.T, emb4)
    return out_t.transpose(2, 0, 1)


# speedup vs baseline: 3.4825x; 3.4825x over previous
import jax
import jax.numpy as jnp
from jax import lax
from jax.experimental import pallas as pl
from jax.experimental.pallas import tpu as pltpu
from jax.experimental.pallas import tpu_sc as plsc

MULT = 1315423911
DPAD = 512
L = 16


def _scale_body(emb_ref, g_ref, out_ref):
    D = emb_ref.shape[1]
    out_ref[:, :D] = emb_ref[...] * jax.nn.sigmoid(g_ref[...])
    out_ref[:, D:] = jnp.zeros_like(out_ref[:, D:])


def _scale_table(emb, g):
    V, D = emb.shape
    blk = 512
    return pl.pallas_call(
        _scale_body,
        out_shape=jax.ShapeDtypeStruct((V, DPAD), emb.dtype),
        grid=(V // blk,),
        in_specs=[
            pl.BlockSpec((blk, D), lambda i: (i, 0)),
            pl.BlockSpec((1, D), lambda i: (0, 0)),
        ],
        out_specs=pl.BlockSpec((blk, DPAD), lambda i: (i, 0)),
    )(emb, g.reshape(1, D))


def _make_sc_lookup(TOK, T, V, D):
    info = plsc.get_sparse_core_info()
    NW = info.num_cores * info.num_subcores
    assert TOK % NW == 0
    per_w = TOK // NW
    assert per_w % T == 0
    assert per_w % L == 0 and D % L == 0
    nvec = per_w // L

    CHUNK = 40
    NBUF = 4
    NOUT = 2
    assert per_w % (CHUNK * NBUF) == 0
    nchunks = per_w // CHUNK
    n_outer = nchunks // NBUF

    mesh = plsc.VectorSubcoreMesh(core_axis_name="c", subcore_axis_name="s")

    def body(x_hbm, emb_hbm, out_hbm, x_v, rows, stage, gsems, osems):
        wid = lax.axis_index("c") * info.num_subcores + lax.axis_index("s")
        base = wid * per_w

        pltpu.sync_copy(x_hbm.at[pl.ds(base, per_w)], x_v.at[pl.ds(8, per_w)])

        def hash_body(r, _):
            i = (nvec - 1) - r
            cur = x_v[pl.ds(8 + i * L, L)]
            prv = x_v[pl.ds(7 + i * L, L)]
            pos = i * L + lax.iota(jnp.int32, 16)
            prv = jnp.where(pos % T == 0, 0, prv)
            x_v[pl.ds(8 + i * L, L)] = (prv * MULT + cur) & (V - 1)
            return 0

        lax.fori_loop(0, nvec, hash_body, 0)

        def issue_gather(k, b):
            return pltpu.async_copy(
                emb_hbm.at[x_v.at[pl.ds(8 + k * CHUNK, CHUNK)]], rows[b], gsems[b]
            )

        for b in range(NBUF):
            issue_gather(b, b)

        def repack(b, o):
            def row2(i, _):
                for r in range(2):
                    for j in range(D // L):
                        stage[o][i * 2 + r, pl.ds(j * L, L)] = (
                            rows[b][i * 2 + r, pl.ds(j * L, L)]
                        )
                return 0

            lax.fori_loop(0, CHUNK // 2, row2, 0)

        def outer(p, _):
            for b in range(NBUF):
                k = p * NBUF + b
                o = b % NOUT
                pltpu.make_async_copy(
                    emb_hbm.at[pl.ds(0, CHUNK)], rows[b], gsems[b]
                ).wait()

                @pl.when(k >= NOUT)
                def _():
                    pltpu.make_async_copy(
                        stage[o], out_hbm.at[pl.ds(0, CHUNK)], osems[o]
                    ).wait()

                repack(b, o)
                pltpu.async_copy(
                    stage[o], out_hbm.at[pl.ds(base + k * CHUNK, CHUNK)], osems[o]
                )

                @pl.when(k + NBUF < nchunks)
                def _():
                    issue_gather(k + NBUF, b)

            return 0

        lax.fori_loop(0, n_outer, outer, 0)

        for o in range(NOUT):
            pltpu.make_async_copy(
                stage[o], out_hbm.at[pl.ds(0, CHUNK)], osems[o]
            ).wait()

    scratch = [
        pltpu.VMEM((per_w + 8,), jnp.int32),
        [pltpu.VMEM((CHUNK, DPAD), jnp.float32) for _ in range(NBUF)],
        [pltpu.VMEM((CHUNK, D), jnp.float32) for _ in range(NOUT)],
        [pltpu.SemaphoreType.DMA for _ in range(NBUF)],
        [pltpu.SemaphoreType.DMA for _ in range(NOUT)],
    ]

    return pl.kernel(
        body,
        out_type=jax.ShapeDtypeStruct((TOK, D), jnp.float32),
        mesh=mesh,
        scratch_types=scratch,
        compiler_params=pltpu.CompilerParams(needs_layout_passes=False),
    )


@jax.jit
def kernel(x, emb, g):
    B, T = x.shape
    V, D = emb.shape
    emb_s = _scale_table(emb, g)
    lookup = _make_sc_lookup(B * T, T, V, D)
    out = lookup(x.reshape(-1), emb_s)
    return out.reshape(B, T, D)
